# baseline (speedup 1.0000x reference)
import functools

import jax
import jax.numpy as jnp
from jax import lax
from jax.experimental import pallas as pl
from jax.experimental.pallas import tpu as pltpu
from jax.experimental.pallas import tpu_sc as plsc

N = 10000
E = 320000
HID = 128
H = 4
D = HID // H
TEMP = float(HID) ** (-0.5)

NHALF = 4
EH = E // NHALF
BE = 4000
GRID_EH = EH // BE
BN = 2000
CB = 40
EDGES_PER_TILE = EH // 16
CHUNKS_PER_TILE = EDGES_PER_TILE // CB
ROWS_PER_TILE = 624
ROWS_TAIL_OFF = 16 * ROWS_PER_TILE
ROWS_TAIL = N - ROWS_TAIL_OFF


def _edge_body(k_ref, q_ref, v_ref, hm_ref, em_ref, w_ref, e_ref):
    p = k_ref[...] * q_ref[...]
    s = lax.dot(p, hm_ref[...], precision=lax.Precision.HIGHEST,
                preferred_element_type=jnp.float32) * TEMP
    s = jnp.where(s >= 0.0, s, 0.2 * s)
    s = jnp.minimum(s, 70.0)
    e = jnp.exp(s)
    er = lax.dot(e, em_ref[...], preferred_element_type=jnp.float32)
    e_ref[...] = er
    w_ref[...] = v_ref[...] * er


def _scatter_body(ebase, w_hbm, e_hbm, dst_hbm, zn_hbm, zd_hbm,
                  num_hbm, den_hbm,
                  acc, vma, vmb, iva, ivb, sema, semb, semia, semib):
    c = lax.axis_index("c")
    s = lax.axis_index("s")
    r0 = s * ROWS_PER_TILE

    def init_from(src_ref):
        pltpu.sync_copy(src_ref.at[pl.ds(r0, ROWS_PER_TILE), :],
                        acc.at[pl.ds(r0, ROWS_PER_TILE), :])

        @pl.when(s == 15)
        def _init_tail():
            pltpu.sync_copy(src_ref.at[pl.ds(ROWS_TAIL_OFF, ROWS_TAIL), :],
                            acc.at[pl.ds(ROWS_TAIL_OFF, ROWS_TAIL), :])

    @pl.when(c == 0)
    def _seed_num():
        init_from(zn_hbm)

    @pl.when(c == 1)
    def _seed_den():
        init_from(zd_hbm)

    plsc.subcore_barrier()
    base = s * EDGES_PER_TILE

    def chunk_from(src_hbm):
        def load(ci, vm, iv, sem, semi):
            off = base + ci * CB
            pltpu.async_copy(src_hbm.at[pl.ds(off, CB), :], vm, sem)
            pltpu.async_copy(dst_hbm.at[pl.ds(ebase + off, CB)], iv, semi)

        def waitbufs(vm, iv, sem, semi):
            pltpu.make_async_copy(src_hbm.at[pl.ds(0, CB), :], vm, sem).wait()
            pltpu.make_async_copy(dst_hbm.at[pl.ds(0, CB)], iv, semi).wait()

        load(0, vma, iva, sema, semia)
        load(1, vmb, ivb, semb, semib)

        def step(k, carry):
            i0 = 2 * k
            waitbufs(vma, iva, sema, semia)
            pltpu.sync_copy(vma, acc.at[iva], add=True)

            @pl.when(i0 + 2 < CHUNKS_PER_TILE)
            def _():
                load(i0 + 2, vma, iva, sema, semia)

            waitbufs(vmb, ivb, semb, semib)
            pltpu.sync_copy(vmb, acc.at[ivb], add=True)

            @pl.when(i0 + 3 < CHUNKS_PER_TILE)
            def _():
                load(i0 + 3, vmb, ivb, semb, semib)

            return carry

        lax.fori_loop(0, CHUNKS_PER_TILE // 2, step, 0)

        if CHUNKS_PER_TILE % 2:
            waitbufs(vma, iva, sema, semia)
            pltpu.sync_copy(vma, acc.at[iva], add=True)

    @pl.when(c == 0)
    def _num():
        chunk_from(w_hbm)

    @pl.when(c == 1)
    def _den():
        chunk_from(e_hbm)

    plsc.subcore_barrier()

    def dump(dst_ref):
        pltpu.sync_copy(acc.at[pl.ds(r0, ROWS_PER_TILE), :],
                        dst_ref.at[pl.ds(r0, ROWS_PER_TILE), :])

        @pl.when(s == 15)
        def _dump_tail():
            pltpu.sync_copy(acc.at[pl.ds(ROWS_TAIL_OFF, ROWS_TAIL), :],
                            dst_ref.at[pl.ds(ROWS_TAIL_OFF, ROWS_TAIL), :])

    @pl.when(c == 0)
    def _dump_num():
        dump(num_hbm)

    @pl.when(c == 1)
    def _dump_den():
        dump(den_hbm)


def _divide_body(n_ref, d_ref, o_ref):
    den = d_ref[...]
    o_ref[...] = jnp.where(den > 0.0, n_ref[...] / den, 0.0)


def kernel(edge_index, keys, queries, values):
    dst = edge_index[1]
    expand = jnp.repeat(jnp.eye(H, dtype=jnp.float32), D, axis=1)
    hmat = expand.T
    zeros = jnp.zeros((N, HID), jnp.float32)

    def edge_half(h):
        return pl.pallas_call(
            _edge_body,
            grid=(GRID_EH,),
            in_specs=[
                pl.BlockSpec((BE, HID), lambda i, h=h: (i + h * GRID_EH, 0)),
                pl.BlockSpec((BE, HID), lambda i, h=h: (i + h * GRID_EH, 0)),
                pl.BlockSpec((BE, HID), lambda i, h=h: (i + h * GRID_EH, 0)),
                pl.BlockSpec((HID, H), lambda i: (0, 0)),
                pl.BlockSpec((H, HID), lambda i: (0, 0)),
            ],
            out_specs=[
                pl.BlockSpec((BE, HID), lambda i: (i, 0)),
                pl.BlockSpec((BE, HID), lambda i: (i, 0)),
            ],
            out_shape=[
                jax.ShapeDtypeStruct((EH, HID), jnp.float32),
                jax.ShapeDtypeStruct((EH, HID), jnp.float32),
            ],
        )(keys, queries, values, hmat, expand)

    def scatter_half(h, weighted, erep, init_num, init_den):
        sc = pl.kernel(
            functools.partial(_scatter_body, h * EH),
            out_type=[
                jax.ShapeDtypeStruct((N, HID), jnp.float32),
                jax.ShapeDtypeStruct((N, HID), jnp.float32),
            ],
            mesh=plsc.VectorSubcoreMesh(core_axis_name="c",
                                        subcore_axis_name="s"),
            scratch_types=[
                pltpu.VMEM_SHARED((N, HID), jnp.float32),
                pltpu.VMEM((CB, HID), jnp.float32),
                pltpu.VMEM((CB, HID), jnp.float32),
                pltpu.VMEM((CB,), jnp.int32),
                pltpu.VMEM((CB,), jnp.int32),
                pltpu.SemaphoreType.DMA,
                pltpu.SemaphoreType.DMA,
                pltpu.SemaphoreType.DMA,
                pltpu.SemaphoreType.DMA,
            ],
        )
        return sc(weighted, erep, dst, init_num, init_den)

    num, den = zeros, zeros
    for h in range(NHALF):
        w, e = edge_half(h)
        num, den = scatter_half(h, w, e, num, den)

    out = pl.pallas_call(
        _divide_body,
        grid=(N // BN,),
        in_specs=[
            pl.BlockSpec((BN, HID), lambda i: (i, 0)),
            pl.BlockSpec((BN, HID), lambda i: (i, 0)),
        ],
        out_specs=pl.BlockSpec((BN, HID), lambda i: (i, 0)),
        out_shape=jax.ShapeDtypeStruct((N, HID), jnp.float32),
    )(num, den)
    return out


# speedup vs baseline: 1.1817x; 1.1817x over previous
import functools

import jax
import jax.numpy as jnp
from jax import lax
from jax.experimental import pallas as pl
from jax.experimental.pallas import tpu as pltpu
from jax.experimental.pallas import tpu_sc as plsc

N = 10000
E = 320000
HID = 128
H = 4
D = HID // H
TEMP = float(HID) ** (-0.5)

NHALF = 2
EH = E // NHALF
BE = 4000
GRID_EH = EH // BE
BN = 2000
CB = 80
EDGES_PER_TILE = EH // 16
CHUNKS_PER_TILE = EDGES_PER_TILE // CB
ROWS_PER_TILE = 624
ROWS_TAIL_OFF = 16 * ROWS_PER_TILE
ROWS_TAIL = N - ROWS_TAIL_OFF


def _edge_body(k_ref, q_ref, v_ref, hm_ref, em_ref, w_ref, e_ref):
    p = k_ref[...] * q_ref[...]
    s = lax.dot(p, hm_ref[...], precision=lax.Precision.HIGHEST,
                preferred_element_type=jnp.float32) * TEMP
    s = jnp.where(s >= 0.0, s, 0.2 * s)
    s = jnp.minimum(s, 70.0)
    e = jnp.exp(s)
    er = lax.dot(e, em_ref[...], preferred_element_type=jnp.float32)
    e_ref[...] = er
    w_ref[...] = v_ref[...] * er


def _scatter_body(ebase, w_hbm, e_hbm, dst_hbm, zn_hbm, zd_hbm,
                  num_hbm, den_hbm,
                  acc, vma, vmb, iva, ivb, sema, semb, semia, semib):
    c = lax.axis_index("c")
    s = lax.axis_index("s")
    r0 = s * ROWS_PER_TILE

    def init_from(src_ref):
        pltpu.sync_copy(src_ref.at[pl.ds(r0, ROWS_PER_TILE), :],
                        acc.at[pl.ds(r0, ROWS_PER_TILE), :])

        @pl.when(s == 15)
        def _init_tail():
            pltpu.sync_copy(src_ref.at[pl.ds(ROWS_TAIL_OFF, ROWS_TAIL), :],
                            acc.at[pl.ds(ROWS_TAIL_OFF, ROWS_TAIL), :])

    @pl.when(c == 0)
    def _seed_num():
        init_from(zn_hbm)

    @pl.when(c == 1)
    def _seed_den():
        init_from(zd_hbm)

    plsc.subcore_barrier()
    base = s * EDGES_PER_TILE

    def chunk_from(src_hbm):
        def load(ci, vm, iv, sem, semi):
            off = base + ci * CB
            pltpu.async_copy(src_hbm.at[pl.ds(off, CB), :], vm, sem)
            pltpu.async_copy(dst_hbm.at[pl.ds(ebase + off, CB)], iv, semi)

        def waitbufs(vm, iv, sem, semi):
            pltpu.make_async_copy(src_hbm.at[pl.ds(0, CB), :], vm, sem).wait()
            pltpu.make_async_copy(dst_hbm.at[pl.ds(0, CB)], iv, semi).wait()

        load(0, vma, iva, sema, semia)
        load(1, vmb, ivb, semb, semib)

        def step(k, carry):
            i0 = 2 * k
            waitbufs(vma, iva, sema, semia)
            pltpu.sync_copy(vma, acc.at[iva], add=True)

            @pl.when(i0 + 2 < CHUNKS_PER_TILE)
            def _():
                load(i0 + 2, vma, iva, sema, semia)

            waitbufs(vmb, ivb, semb, semib)
            pltpu.sync_copy(vmb, acc.at[ivb], add=True)

            @pl.when(i0 + 3 < CHUNKS_PER_TILE)
            def _():
                load(i0 + 3, vmb, ivb, semb, semib)

            return carry

        lax.fori_loop(0, CHUNKS_PER_TILE // 2, step, 0)

        if CHUNKS_PER_TILE % 2:
            waitbufs(vma, iva, sema, semia)
            pltpu.sync_copy(vma, acc.at[iva], add=True)

    @pl.when(c == 0)
    def _num():
        chunk_from(w_hbm)

    @pl.when(c == 1)
    def _den():
        chunk_from(e_hbm)

    plsc.subcore_barrier()

    def dump(dst_ref):
        pltpu.sync_copy(acc.at[pl.ds(r0, ROWS_PER_TILE), :],
                        dst_ref.at[pl.ds(r0, ROWS_PER_TILE), :])

        @pl.when(s == 15)
        def _dump_tail():
            pltpu.sync_copy(acc.at[pl.ds(ROWS_TAIL_OFF, ROWS_TAIL), :],
                            dst_ref.at[pl.ds(ROWS_TAIL_OFF, ROWS_TAIL), :])

    @pl.when(c == 0)
    def _dump_num():
        dump(num_hbm)

    @pl.when(c == 1)
    def _dump_den():
        dump(den_hbm)


def _divide_body(n_ref, d_ref, o_ref):
    den = d_ref[...]
    o_ref[...] = jnp.where(den > 0.0, n_ref[...] / den, 0.0)


def kernel(edge_index, keys, queries, values):
    dst = edge_index[1]
    expand = jnp.repeat(jnp.eye(H, dtype=jnp.float32), D, axis=1)
    hmat = expand.T
    zeros = jnp.zeros((N, HID), jnp.float32)

    def edge_half(h):
        return pl.pallas_call(
            _edge_body,
            grid=(GRID_EH,),
            in_specs=[
                pl.BlockSpec((BE, HID), lambda i, h=h: (i + h * GRID_EH, 0)),
                pl.BlockSpec((BE, HID), lambda i, h=h: (i + h * GRID_EH, 0)),
                pl.BlockSpec((BE, HID), lambda i, h=h: (i + h * GRID_EH, 0)),
                pl.BlockSpec((HID, H), lambda i: (0, 0)),
                pl.BlockSpec((H, HID), lambda i: (0, 0)),
            ],
            out_specs=[
                pl.BlockSpec((BE, HID), lambda i: (i, 0)),
                pl.BlockSpec((BE, HID), lambda i: (i, 0)),
            ],
            out_shape=[
                jax.ShapeDtypeStruct((EH, HID), jnp.float32),
                jax.ShapeDtypeStruct((EH, HID), jnp.float32),
            ],
        )(keys, queries, values, hmat, expand)

    def scatter_half(h, weighted, erep, init_num, init_den):
        sc = pl.kernel(
            functools.partial(_scatter_body, h * EH),
            out_type=[
                jax.ShapeDtypeStruct((N, HID), jnp.float32),
                jax.ShapeDtypeStruct((N, HID), jnp.float32),
            ],
            mesh=plsc.VectorSubcoreMesh(core_axis_name="c",
                                        subcore_axis_name="s"),
            scratch_types=[
                pltpu.VMEM_SHARED((N, HID), jnp.float32),
                pltpu.VMEM((CB, HID), jnp.float32),
                pltpu.VMEM((CB, HID), jnp.float32),
                pltpu.VMEM((CB,), jnp.int32),
                pltpu.VMEM((CB,), jnp.int32),
                pltpu.SemaphoreType.DMA,
                pltpu.SemaphoreType.DMA,
                pltpu.SemaphoreType.DMA,
                pltpu.SemaphoreType.DMA,
            ],
        )
        return sc(weighted, erep, dst, init_num, init_den)

    num, den = zeros, zeros
    for h in range(NHALF):
        w, e = edge_half(h)
        num, den = scatter_half(h, w, e, num, den)

    out = pl.pallas_call(
        _divide_body,
        grid=(N // BN,),
        in_specs=[
            pl.BlockSpec((BN, HID), lambda i: (i, 0)),
            pl.BlockSpec((BN, HID), lambda i: (i, 0)),
        ],
        out_specs=pl.BlockSpec((BN, HID), lambda i: (i, 0)),
        out_shape=jax.ShapeDtypeStruct((N, HID), jnp.float32),
    )(num, den)
    return out
